# baseline (speedup 1.0000x reference)
import jax
import jax.numpy as jnp
from jax.experimental import pallas as pl
from jax.experimental.pallas import tpu as pltpu

M_BLK = 4000
NSPLIT = 8


def _copy_chunks(src, dst, sem, nsplit):
    n = src.shape[0]
    rows = (n // nsplit) // 8 * 8
    for i in range(nsplit):
        lo = i * rows
        sz = rows if i < nsplit - 1 else n - lo
        yield src.at[pl.ds(lo, sz)], dst.at[pl.ds(lo, sz)], sem


def _body(p_ref, q_ref, pq2_ref, x_ref, w_ref, w_any,
          mm_ref, op_ref, oq_ref, opq2_ref, ow_ref,
          sem_p, sem_q, sem_pq2, sem_w):
    i = pl.program_id(0)
    nsteps = pl.num_programs(0)

    @pl.when(i == 0)
    def _start_copies():
        for s, d, sem in _copy_chunks(p_ref, op_ref, sem_p, NSPLIT):
            pltpu.async_copy(s, d, sem)
        for s, d, sem in _copy_chunks(pq2_ref, opq2_ref, sem_pq2, NSPLIT):
            pltpu.async_copy(s, d, sem)
        for s, d, sem in _copy_chunks(q_ref, oq_ref, sem_q, 2):
            pltpu.async_copy(s, d, sem)
        pltpu.async_copy(w_any, ow_ref, sem_w)

    mm_ref[...] = jnp.dot(x_ref[...], w_ref[...],
                          preferred_element_type=jnp.float32)

    @pl.when(i == nsteps - 1)
    def _drain_copies():
        for s, d, sem in _copy_chunks(p_ref, op_ref, sem_p, NSPLIT):
            pltpu.make_async_copy(s, d, sem).wait()
        for s, d, sem in _copy_chunks(pq2_ref, opq2_ref, sem_pq2, NSPLIT):
            pltpu.make_async_copy(s, d, sem).wait()
        for s, d, sem in _copy_chunks(q_ref, oq_ref, sem_q, 2):
            pltpu.make_async_copy(s, d, sem).wait()
        pltpu.make_async_copy(w_any, ow_ref, sem_w).wait()


def kernel(P, Q, PQ2, item_content, W):
    M, K = item_content.shape
    N = W.shape[1]
    grid = (M // M_BLK,)
    any_spec = pl.BlockSpec(memory_space=pl.ANY)
    mm, oP, oQ, oPQ2, oW = pl.pallas_call(
        _body,
        grid=grid,
        in_specs=[
            any_spec,
            any_spec,
            any_spec,
            pl.BlockSpec((M_BLK, K), lambda i: (i, 0)),
            pl.BlockSpec((K, N), lambda i: (0, 0)),
            any_spec,
        ],
        out_specs=[
            pl.BlockSpec((M_BLK, N), lambda i: (i, 0)),
            any_spec,
            any_spec,
            any_spec,
            any_spec,
        ],
        out_shape=[
            jax.ShapeDtypeStruct((M, N), jnp.float32),
            jax.ShapeDtypeStruct(P.shape, P.dtype),
            jax.ShapeDtypeStruct(Q.shape, Q.dtype),
            jax.ShapeDtypeStruct(PQ2.shape, PQ2.dtype),
            jax.ShapeDtypeStruct(W.shape, W.dtype),
        ],
        scratch_shapes=[
            pltpu.SemaphoreType.DMA,
            pltpu.SemaphoreType.DMA,
            pltpu.SemaphoreType.DMA,
            pltpu.SemaphoreType.DMA,
        ],
    )(P, Q, PQ2, item_content, W, W)
    return (oP, oQ, oPQ2, mm, oW)


# speedup vs baseline: 15.4652x; 15.4652x over previous
import jax
import jax.numpy as jnp
from jax.experimental import pallas as pl
from jax.experimental.pallas import tpu as pltpu

M_BLK = 1000
U_BLK = 10000


def _body(x_ref, w_ref, q_ref, p_ref, pq2_ref,
          mm_ref, oq_ref, op_ref, opq2_ref, ow_ref):
    mm_ref[...] = jnp.dot(x_ref[...], w_ref[...],
                          preferred_element_type=jnp.float32)
    oq_ref[...] = q_ref[...]
    op_ref[...] = p_ref[...]
    opq2_ref[...] = pq2_ref[...]

    @pl.when(pl.program_id(0) == 0)
    def _():
        ow_ref[...] = w_ref[...]


def kernel(P, Q, PQ2, item_content, W):
    M, K = item_content.shape
    N = W.shape[1]
    U = P.shape[0]
    grid = (M // M_BLK,)
    mm, oQ, oP, oPQ2, oW = pl.pallas_call(
        _body,
        grid=grid,
        in_specs=[
            pl.BlockSpec((M_BLK, K), lambda i: (i, 0)),
            pl.BlockSpec((K, N), lambda i: (0, 0)),
            pl.BlockSpec((M_BLK, N), lambda i: (i, 0)),
            pl.BlockSpec((U_BLK, N), lambda i: (i, 0)),
            pl.BlockSpec((U_BLK, N), lambda i: (i, 0)),
        ],
        out_specs=[
            pl.BlockSpec((M_BLK, N), lambda i: (i, 0)),
            pl.BlockSpec((M_BLK, N), lambda i: (i, 0)),
            pl.BlockSpec((U_BLK, N), lambda i: (i, 0)),
            pl.BlockSpec((U_BLK, N), lambda i: (i, 0)),
            pl.BlockSpec((K, N), lambda i: (0, 0)),
        ],
        out_shape=[
            jax.ShapeDtypeStruct((M, N), jnp.float32),
            jax.ShapeDtypeStruct(Q.shape, Q.dtype),
            jax.ShapeDtypeStruct(P.shape, P.dtype),
            jax.ShapeDtypeStruct(PQ2.shape, PQ2.dtype),
            jax.ShapeDtypeStruct(W.shape, W.dtype),
        ],
    )(item_content, W, Q, P, PQ2)
    return (oP, oQ, oPQ2, mm, oW)


# speedup vs baseline: 75.3942x; 4.8751x over previous
import jax
import jax.numpy as jnp
from jax.experimental import pallas as pl
from jax.experimental.pallas import tpu as pltpu

M_BLK = 10000


def _matmul_body(x_ref, w_ref, o_ref):
    o_ref[...] = jnp.dot(x_ref[...], w_ref[...],
                         preferred_element_type=jnp.float32)


def _content_matmul(item_content, W):
    M, K = item_content.shape
    N = W.shape[1]
    grid = (M // M_BLK,)
    return pl.pallas_call(
        _matmul_body,
        grid=grid,
        in_specs=[
            pl.BlockSpec((M_BLK, K), lambda i: (i, 0)),
            pl.BlockSpec((K, N), lambda i: (0, 0)),
        ],
        out_specs=pl.BlockSpec((M_BLK, N), lambda i: (i, 0)),
        out_shape=jax.ShapeDtypeStruct((M, N), jnp.float32),
        compiler_params=pltpu.CompilerParams(
            dimension_semantics=("arbitrary",),
        ),
    )(item_content, W)


def kernel(P, Q, PQ2, item_content, W):
    item_emb2 = _content_matmul(item_content, W)
    return (P, Q, PQ2, item_emb2, W)
